# baseline (speedup 1.0000x reference)
import functools

import jax
import jax.numpy as jnp
from jax.experimental import pallas as pl

_NB = 512


def _vq_body(z_ref, cb_ref, out_ref, idx_ref, loss_ref):
    first = (pl.program_id(0) == 0) & (pl.program_id(1) == 0)
    xb = z_ref[0].T
    cb = cb_ref[...]
    n, _ = xb.shape
    k = cb.shape[0]

    mm = jax.lax.dot_general(
        xb, cb, (((1,), (1,)), ((), ())),
        preferred_element_type=jnp.float32)
    x2 = jnp.sum(xb * xb, axis=1, keepdims=True)
    c2 = jnp.sum(cb * cb, axis=1)
    dist = (x2 - 2.0 * mm) + c2[None, :]

    minval = jnp.min(dist, axis=1, keepdims=True)
    iota = jax.lax.broadcasted_iota(jnp.int32, (n, k), 1)
    idx = jnp.min(jnp.where(dist == minval, iota, k), axis=1)
    idx_ref[0, 0, :] = idx

    onehot = (iota == idx[:, None]).astype(jnp.bfloat16)
    outb = jax.lax.dot_general(
        cb.astype(jnp.bfloat16), onehot, (((0,), (1,)), ((), ())),
        preferred_element_type=jnp.float32)
    out_ref[0] = outb

    psum = jnp.sum(minval).reshape(1, 1)

    @pl.when(first)
    def _():
        loss_ref[...] = jnp.zeros((1, 1), jnp.float32)
    loss_ref[...] += psum


@functools.partial(jax.jit, static_argnames=("interpret",))
def kernel(z, codebook, interpret=False):
    B, d, N = z.shape
    K = codebook.shape[0]

    out, idx3, loss_sum = pl.pallas_call(
        _vq_body,
        grid=(B, N // _NB),
        in_specs=[
            pl.BlockSpec((1, d, _NB), lambda b, j: (b, 0, j)),
            pl.BlockSpec((K, d), lambda b, j: (0, 0)),
        ],
        out_specs=[
            pl.BlockSpec((1, d, _NB), lambda b, j: (b, 0, j)),
            pl.BlockSpec((1, 1, _NB), lambda b, j: (b, 0, j)),
            pl.BlockSpec((1, 1), lambda b, j: (0, 0)),
        ],
        out_shape=[
            jax.ShapeDtypeStruct((B, d, N), jnp.float32),
            jax.ShapeDtypeStruct((B, 1, N), jnp.int32),
            jax.ShapeDtypeStruct((1, 1), jnp.float32),
        ],
        interpret=interpret,
    )(z, codebook)

    commit_loss = 0.25 * loss_sum[0, 0] / (B * N * d)
    return out, idx3.reshape(B, N), commit_loss


# speedup vs baseline: 1.1231x; 1.1231x over previous
import functools

import jax
import jax.numpy as jnp
from jax.experimental import pallas as pl


def _vq_body(z_ref, cb_ref, cbm2_ref, out_ref, idx_ref, loss_ref):
    b = pl.program_id(0)
    xb = z_ref[0].T
    cb = cb_ref[...]
    n = xb.shape[0]
    k = cb.shape[0]

    mm2 = jax.lax.dot_general(
        xb, cbm2_ref[...], (((1,), (1,)), ((), ())),
        preferred_element_type=jnp.float32)
    x2 = jnp.sum(xb * xb, axis=1, keepdims=True)
    c2 = jnp.sum(cb * cb, axis=1)
    dist = (x2 + mm2) + c2[None, :]

    ch = 256
    nch = k // ch
    iota_c = jax.lax.broadcasted_iota(jnp.int32, (n, ch), 1)
    rv = dist[:, 0:ch]
    for c in range(1, nch):
        rv = jnp.minimum(rv, dist[:, c * ch:(c + 1) * ch])
    minval = jnp.min(rv, axis=1, keepdims=True)
    ri = jnp.where(dist[:, 0:ch] == minval, iota_c, k)
    for c in range(1, nch):
        cand = jnp.where(dist[:, c * ch:(c + 1) * ch] == minval,
                         iota_c + c * ch, k)
        ri = jnp.minimum(ri, cand)
    idx = jnp.min(ri, axis=1)
    idx_ref[0, 0, :] = idx

    iota = jax.lax.broadcasted_iota(jnp.int32, (n, k), 1)
    onehot = (iota == idx[:, None]).astype(jnp.bfloat16)
    outb = jax.lax.dot_general(
        cb.astype(jnp.bfloat16), onehot, (((0,), (1,)), ((), ())),
        preferred_element_type=jnp.float32)
    out_ref[0] = outb

    psum = jnp.sum(minval).reshape(1, 1)

    @pl.when(b == 0)
    def _():
        loss_ref[...] = jnp.zeros((1, 1), jnp.float32)
    loss_ref[...] += psum


@functools.partial(jax.jit, static_argnames=("interpret",))
def kernel(z, codebook, interpret=False):
    B, d, N = z.shape
    K = codebook.shape[0]

    out, idx3, loss_sum = pl.pallas_call(
        _vq_body,
        grid=(B,),
        in_specs=[
            pl.BlockSpec((1, d, N), lambda b: (b, 0, 0)),
            pl.BlockSpec((K, d), lambda b: (0, 0)),
            pl.BlockSpec((K, d), lambda b: (0, 0)),
        ],
        out_specs=[
            pl.BlockSpec((1, d, N), lambda b: (b, 0, 0)),
            pl.BlockSpec((1, 1, N), lambda b: (b, 0, 0)),
            pl.BlockSpec((1, 1), lambda b: (0, 0)),
        ],
        out_shape=[
            jax.ShapeDtypeStruct((B, d, N), jnp.float32),
            jax.ShapeDtypeStruct((B, 1, N), jnp.int32),
            jax.ShapeDtypeStruct((1, 1), jnp.float32),
        ],
        interpret=interpret,
    )(z, codebook, codebook * -2.0)

    commit_loss = 0.25 * loss_sum[0, 0] / (B * N * d)
    return out, idx3.reshape(B, N), commit_loss


# speedup vs baseline: 1.1232x; 1.0001x over previous
import functools

import jax
import jax.numpy as jnp
from jax.experimental import pallas as pl


def _vq_body(z_ref, cb_ref, cbm2_ref, out_ref, idx_ref, loss_ref):
    b = pl.program_id(0)
    xb = z_ref[0].T
    cb = cb_ref[...]
    n = xb.shape[0]
    k = cb.shape[0]

    mm2 = jax.lax.dot_general(
        xb.astype(jnp.bfloat16), cbm2_ref[...], (((1,), (1,)), ((), ())),
        preferred_element_type=jnp.float32)
    x2 = jnp.sum(xb * xb, axis=1, keepdims=True)
    c2 = jnp.sum(cb * cb, axis=1)
    dist = (x2 + mm2) + c2[None, :]

    ch = 256
    iota_c = jax.lax.broadcasted_iota(jnp.int32, (n, ch), 1)
    rv = dist[:, 0:ch]
    for c in range(1, k // ch):
        rv = jnp.minimum(rv, dist[:, c * ch:(c + 1) * ch])
    minval = jnp.min(rv, axis=1, keepdims=True)
    ri = jnp.where(dist[:, 0:ch] == minval, iota_c, k)
    for c in range(1, k // ch):
        ri = jnp.minimum(ri, jnp.where(
            dist[:, c * ch:(c + 1) * ch] == minval, iota_c + c * ch, k))
    idx = jnp.min(ri, axis=1)
    idx_ref[0, 0, :] = idx

    iota = jax.lax.broadcasted_iota(jnp.int32, (n, k), 1)
    onehot = (iota == idx[:, None]).astype(jnp.bfloat16)
    outb = jax.lax.dot_general(
        cb.astype(jnp.bfloat16), onehot, (((0,), (1,)), ((), ())),
        preferred_element_type=jnp.float32)
    out_ref[0] = outb

    psum = jnp.sum(minval).reshape(1, 1)

    @pl.when(b == 0)
    def _():
        loss_ref[...] = jnp.zeros((1, 1), jnp.float32)
    loss_ref[...] += psum


@functools.partial(jax.jit, static_argnames=("interpret",))
def kernel(z, codebook, interpret=False):
    B, d, N = z.shape
    K = codebook.shape[0]

    out, idx3, loss_sum = pl.pallas_call(
        _vq_body,
        grid=(B,),
        in_specs=[
            pl.BlockSpec((1, d, N), lambda b: (b, 0, 0)),
            pl.BlockSpec((K, d), lambda b: (0, 0)),
            pl.BlockSpec((K, d), lambda b: (0, 0)),
        ],
        out_specs=[
            pl.BlockSpec((1, d, N), lambda b: (b, 0, 0)),
            pl.BlockSpec((1, 1, N), lambda b: (b, 0, 0)),
            pl.BlockSpec((1, 1), lambda b: (0, 0)),
        ],
        out_shape=[
            jax.ShapeDtypeStruct((B, d, N), jnp.float32),
            jax.ShapeDtypeStruct((B, 1, N), jnp.int32),
            jax.ShapeDtypeStruct((1, 1), jnp.float32),
        ],
        interpret=interpret,
    )(z, codebook, (codebook * -2.0).astype(jnp.bfloat16))

    commit_loss = 0.25 * loss_sum[0, 0] / (B * N * d)
    return out, idx3.reshape(B, N), commit_loss


# speedup vs baseline: 1.1292x; 1.0053x over previous
import functools

import jax
import jax.numpy as jnp
from jax.experimental import pallas as pl


def _vq_body(z_ref, cb_ref, cbm2_ref, out_ref, idx_ref, loss_ref):
    b = pl.program_id(0)
    xb = z_ref[0].T
    cb = cb_ref[...]
    n = xb.shape[0]
    k = cb.shape[0]

    mm2 = jax.lax.dot_general(
        xb.astype(jnp.bfloat16), cbm2_ref[...], (((1,), (1,)), ((), ())),
        preferred_element_type=jnp.float32)
    x2 = jnp.sum(xb * xb, axis=1, keepdims=True)
    c2 = jnp.sum(cb * cb, axis=1)

    ch = 512
    nch = k // ch

    def dchunk(c):
        sl = slice(c * ch, (c + 1) * ch)
        return (x2 + mm2[:, sl]) + c2[None, sl]

    iota_c = jax.lax.broadcasted_iota(jnp.int32, (n, ch), 1)
    rv = dchunk(0)
    for c in range(1, nch):
        rv = jnp.minimum(rv, dchunk(c))
    minval = jnp.min(rv, axis=1, keepdims=True)
    ri = jnp.where(dchunk(0) == minval, iota_c, k)
    for c in range(1, nch):
        ri = jnp.minimum(ri, jnp.where(
            dchunk(c) == minval, iota_c + c * ch, k))
    idx = jnp.min(ri, axis=1)
    idx_ref[0, 0, :] = idx

    iota = jax.lax.broadcasted_iota(jnp.int32, (n, k), 1)
    onehot = (iota == idx[:, None]).astype(jnp.bfloat16)
    outb = jax.lax.dot_general(
        cb.astype(jnp.bfloat16), onehot, (((0,), (1,)), ((), ())),
        preferred_element_type=jnp.float32)
    out_ref[0] = outb

    psum = jnp.sum(minval).reshape(1, 1)

    @pl.when(b == 0)
    def _():
        loss_ref[...] = jnp.zeros((1, 1), jnp.float32)
    loss_ref[...] += psum


@functools.partial(jax.jit, static_argnames=("interpret",))
def kernel(z, codebook, interpret=False):
    B, d, N = z.shape
    K = codebook.shape[0]

    out, idx3, loss_sum = pl.pallas_call(
        _vq_body,
        grid=(B,),
        in_specs=[
            pl.BlockSpec((1, d, N), lambda b: (b, 0, 0)),
            pl.BlockSpec((K, d), lambda b: (0, 0)),
            pl.BlockSpec((K, d), lambda b: (0, 0)),
        ],
        out_specs=[
            pl.BlockSpec((1, d, N), lambda b: (b, 0, 0)),
            pl.BlockSpec((1, 1, N), lambda b: (b, 0, 0)),
            pl.BlockSpec((1, 1), lambda b: (0, 0)),
        ],
        out_shape=[
            jax.ShapeDtypeStruct((B, d, N), jnp.float32),
            jax.ShapeDtypeStruct((B, 1, N), jnp.int32),
            jax.ShapeDtypeStruct((1, 1), jnp.float32),
        ],
        interpret=interpret,
    )(z, codebook, (codebook * -2.0).astype(jnp.bfloat16))

    commit_loss = 0.25 * loss_sum[0, 0] / (B * N * d)
    return out, idx3.reshape(B, N), commit_loss


# speedup vs baseline: 1.1346x; 1.0048x over previous
import jax
import jax.numpy as jnp
from jax.experimental import pallas as pl


def _vq_body(z_ref, cb_ref, cbm2_ref, out_ref, idx_ref, loss_ref):
    b = pl.program_id(0)
    xb = z_ref[0].T
    cb = cb_ref[...]
    n = xb.shape[0]
    k = cb.shape[0]

    mm2 = jax.lax.dot_general(
        xb.astype(jnp.bfloat16), cbm2_ref[...], (((1,), (1,)), ((), ())),
        preferred_element_type=jnp.float32)
    x2 = jnp.sum(xb * xb, axis=1, keepdims=True)
    c2 = jnp.sum(cb * cb, axis=1)

    ch = 512
    nch = k // ch

    def dchunk(c):
        sl = slice(c * ch, (c + 1) * ch)
        return (x2 + mm2[:, sl]) + c2[None, sl]

    iota_c = jax.lax.broadcasted_iota(jnp.int32, (n, ch), 1)
    rv = dchunk(0)
    for c in range(1, nch):
        rv = jnp.minimum(rv, dchunk(c))
    minval = jnp.min(rv, axis=1, keepdims=True)
    ri = jnp.where(dchunk(0) == minval, iota_c, k)
    for c in range(1, nch):
        ri = jnp.minimum(ri, jnp.where(
            dchunk(c) == minval, iota_c + c * ch, k))
    idx = jnp.min(ri, axis=1)
    idx_ref[0, 0, :] = idx

    iota = jax.lax.broadcasted_iota(jnp.int32, (n, k), 1)
    onehot = (iota == idx[:, None]).astype(jnp.bfloat16)
    outb = jax.lax.dot_general(
        cb.astype(jnp.bfloat16), onehot, (((0,), (1,)), ((), ())),
        preferred_element_type=jnp.float32)
    out_ref[0] = outb

    psum = jnp.sum(minval).reshape(1, 1)

    @pl.when(b == 0)
    def _():
        loss_ref[...] = jnp.zeros((1, 1), jnp.float32)
    loss_ref[...] += psum


@jax.jit
def kernel(z, codebook):
    B, d, N = z.shape
    K = codebook.shape[0]

    out, idx3, loss_sum = pl.pallas_call(
        _vq_body,
        grid=(B,),
        in_specs=[
            pl.BlockSpec((1, d, N), lambda b: (b, 0, 0)),
            pl.BlockSpec((K, d), lambda b: (0, 0)),
            pl.BlockSpec((K, d), lambda b: (0, 0)),
        ],
        out_specs=[
            pl.BlockSpec((1, d, N), lambda b: (b, 0, 0)),
            pl.BlockSpec((1, 1, N), lambda b: (b, 0, 0)),
            pl.BlockSpec((1, 1), lambda b: (0, 0)),
        ],
        out_shape=[
            jax.ShapeDtypeStruct((B, d, N), jnp.float32),
            jax.ShapeDtypeStruct((B, 1, N), jnp.int32),
            jax.ShapeDtypeStruct((1, 1), jnp.float32),
        ],
    )(z, codebook, (codebook * -2.0).astype(jnp.bfloat16))

    commit_loss = 0.25 * loss_sum[0, 0] / (B * N * d)
    return out, idx3.reshape(B, N), commit_loss


# speedup vs baseline: 1.1357x; 1.0010x over previous
import jax
import jax.numpy as jnp
from jax.experimental import pallas as pl

_UB = 4


def _vq_one(zb, cb16, cbm2, x2col, c2row, out_ref, idx_ref, i):
    d, n = zb.shape
    k = cbm2.shape[0]

    xb16 = zb.astype(jnp.bfloat16).T
    mm2 = jax.lax.dot_general(
        xb16, cbm2, (((1,), (1,)), ((), ())),
        preferred_element_type=jnp.float32)

    ch = 512
    nch = k // ch

    def dchunk(c):
        sl = slice(c * ch, (c + 1) * ch)
        return (x2col + mm2[:, sl]) + c2row[:, sl]

    iota_c = jax.lax.broadcasted_iota(jnp.int32, (n, ch), 1)
    rv = dchunk(0)
    for c in range(1, nch):
        rv = jnp.minimum(rv, dchunk(c))
    minval = jnp.min(rv, axis=1, keepdims=True)
    ri = jnp.where(dchunk(0) == minval, iota_c, k)
    for c in range(1, nch):
        ri = jnp.minimum(ri, jnp.where(
            dchunk(c) == minval, iota_c + c * ch, k))
    idx = jnp.min(ri, axis=1)
    idx_ref[i, 0, :] = idx

    iota = jax.lax.broadcasted_iota(jnp.int32, (n, k), 1)
    onehot = (iota == idx[:, None]).astype(jnp.bfloat16)
    out_ref[i] = jax.lax.dot_general(
        cb16, onehot, (((0,), (1,)), ((), ())),
        preferred_element_type=jnp.float32)

    return jnp.sum(minval)


def _vq_body(z_ref, cb_ref, cbm2_ref, out_ref, idx_ref, loss_ref):
    g = pl.program_id(0)
    cb = cb_ref[...]
    cb16 = cb.astype(jnp.bfloat16)
    cbm2 = cbm2_ref[...]
    c2row = jnp.sum(cb * cb, axis=1)[None, :]

    psum = jnp.zeros((), jnp.float32)
    for i in range(z_ref.shape[0]):
        zb = z_ref[i]
        xb = zb.T
        x2col = jnp.sum(xb * xb, axis=1, keepdims=True)
        psum += _vq_one(zb, cb16, cbm2, x2col, c2row, out_ref, idx_ref, i)

    @pl.when(g == 0)
    def _():
        loss_ref[...] = jnp.zeros((1, 1), jnp.float32)
    loss_ref[...] += psum.reshape(1, 1)


@jax.jit
def kernel(z, codebook):
    B, d, N = z.shape
    K = codebook.shape[0]

    out, idx3, loss_sum = pl.pallas_call(
        _vq_body,
        grid=(B // _UB,),
        in_specs=[
            pl.BlockSpec((_UB, d, N), lambda b: (b, 0, 0)),
            pl.BlockSpec((K, d), lambda b: (0, 0)),
            pl.BlockSpec((K, d), lambda b: (0, 0)),
        ],
        out_specs=[
            pl.BlockSpec((_UB, d, N), lambda b: (b, 0, 0)),
            pl.BlockSpec((_UB, 1, N), lambda b: (b, 0, 0)),
            pl.BlockSpec((1, 1), lambda b: (0, 0)),
        ],
        out_shape=[
            jax.ShapeDtypeStruct((B, d, N), jnp.float32),
            jax.ShapeDtypeStruct((B, 1, N), jnp.int32),
            jax.ShapeDtypeStruct((1, 1), jnp.float32),
        ],
    )(z, codebook, (codebook * -2.0).astype(jnp.bfloat16))

    commit_loss = 0.25 * loss_sum[0, 0] / (B * N * d)
    return out, idx3.reshape(B, N), commit_loss


# speedup vs baseline: 1.1873x; 1.0454x over previous
import jax
import jax.numpy as jnp
from jax.experimental import pallas as pl

_UB = 4


def _vq_finish(mm2, cb16, x2col, c2row, out_ref, idx_ref, i):
    n, k = mm2.shape

    ch = 512
    nch = k // ch

    def dchunk(c):
        sl = slice(c * ch, (c + 1) * ch)
        return (x2col + mm2[:, sl]) + c2row[:, sl]

    iota_c = jax.lax.broadcasted_iota(jnp.int32, (n, ch), 1)
    rv = dchunk(0)
    for c in range(1, nch):
        rv = jnp.minimum(rv, dchunk(c))
    minval = jnp.min(rv, axis=1, keepdims=True)
    ri = jnp.where(dchunk(0) == minval, iota_c, k)
    for c in range(1, nch):
        ri = jnp.minimum(ri, jnp.where(
            dchunk(c) == minval, iota_c + c * ch, k))
    idx = jnp.min(ri, axis=1)
    idx_ref[i, 0, :] = idx

    iota = jax.lax.broadcasted_iota(jnp.int32, (n, k), 1)
    onehot = (iota == idx[:, None]).astype(jnp.bfloat16)
    out_ref[i] = jax.lax.dot_general(
        cb16, onehot, (((0,), (1,)), ((), ())),
        preferred_element_type=jnp.float32)

    return jnp.sum(minval)


def _vq_body(z_ref, cb_ref, cbm2_ref, out_ref, idx_ref, loss_ref):
    g = pl.program_id(0)
    cb = cb_ref[...]
    cb16 = cb.astype(jnp.bfloat16)
    cbm2 = cbm2_ref[...]
    c2row = jnp.sum(cb * cb, axis=1)[None, :]

    ub = z_ref.shape[0]
    x2s, mm2s = [], []
    for i in range(ub):
        zb = z_ref[i]
        xb = zb.T
        x2s.append(jnp.sum(xb * xb, axis=1, keepdims=True))
        mm2s.append(jax.lax.dot_general(
            zb.astype(jnp.bfloat16).T, cbm2, (((1,), (1,)), ((), ())),
            preferred_element_type=jnp.float32))

    psum = jnp.zeros((), jnp.float32)
    for i in range(ub):
        psum += _vq_finish(mm2s[i], cb16, x2s[i], c2row,
                           out_ref, idx_ref, i)

    @pl.when(g == 0)
    def _():
        loss_ref[...] = jnp.zeros((1, 1), jnp.float32)
    loss_ref[...] += psum.reshape(1, 1)


@jax.jit
def kernel(z, codebook):
    B, d, N = z.shape
    K = codebook.shape[0]

    out, idx3, loss_sum = pl.pallas_call(
        _vq_body,
        grid=(B // _UB,),
        in_specs=[
            pl.BlockSpec((_UB, d, N), lambda b: (b, 0, 0)),
            pl.BlockSpec((K, d), lambda b: (0, 0)),
            pl.BlockSpec((K, d), lambda b: (0, 0)),
        ],
        out_specs=[
            pl.BlockSpec((_UB, d, N), lambda b: (b, 0, 0)),
            pl.BlockSpec((_UB, 1, N), lambda b: (b, 0, 0)),
            pl.BlockSpec((1, 1), lambda b: (0, 0)),
        ],
        out_shape=[
            jax.ShapeDtypeStruct((B, d, N), jnp.float32),
            jax.ShapeDtypeStruct((B, 1, N), jnp.int32),
            jax.ShapeDtypeStruct((1, 1), jnp.float32),
        ],
    )(z, codebook, (codebook * -2.0).astype(jnp.bfloat16))

    commit_loss = 0.25 * loss_sum[0, 0] / (B * N * d)
    return out, idx3.reshape(B, N), commit_loss


# speedup vs baseline: 1.1954x; 1.0068x over previous
import jax
import jax.numpy as jnp
from jax.experimental import pallas as pl

_UB = 4


_CH = 512


def _dchunk(mm2, x2col, c2row, c):
    sl = slice(c * _CH, (c + 1) * _CH)
    return (x2col + mm2[:, sl]) + c2row[:, sl]


def _vq_minval(mm2, x2col, c2row):
    k = mm2.shape[1]
    rv = _dchunk(mm2, x2col, c2row, 0)
    for c in range(1, k // _CH):
        rv = jnp.minimum(rv, _dchunk(mm2, x2col, c2row, c))
    return jnp.min(rv, axis=1, keepdims=True)


def _vq_finish(mm2, cb16, x2col, c2row, minval, out_ref, idx_ref, i):
    n, k = mm2.shape

    iota_c = jax.lax.broadcasted_iota(jnp.int32, (n, _CH), 1)
    ri = jnp.where(_dchunk(mm2, x2col, c2row, 0) == minval, iota_c, k)
    for c in range(1, k // _CH):
        ri = jnp.minimum(ri, jnp.where(
            _dchunk(mm2, x2col, c2row, c) == minval, iota_c + c * _CH, k))
    idx = jnp.min(ri, axis=1)
    idx_ref[i, 0, :] = idx

    iota = jax.lax.broadcasted_iota(jnp.int32, (n, k), 1)
    onehot = (iota == idx[:, None]).astype(jnp.bfloat16)
    out_ref[i] = jax.lax.dot_general(
        cb16, onehot, (((0,), (1,)), ((), ())),
        preferred_element_type=jnp.float32)


def _vq_body(z_ref, cb_ref, cbm2_ref, out_ref, idx_ref, loss_ref):
    g = pl.program_id(0)
    cb = cb_ref[...]
    cb16 = cb.astype(jnp.bfloat16)
    cbm2 = cbm2_ref[...]
    c2row = jnp.sum(cb * cb, axis=1)[None, :]

    ub = z_ref.shape[0]
    x2s, mm2s = [], []
    for i in range(ub):
        zb = z_ref[i]
        xb = zb.T
        x2s.append(jnp.sum(xb * xb, axis=1, keepdims=True))
        mm2s.append(jax.lax.dot_general(
            zb.astype(jnp.bfloat16).T, cbm2, (((1,), (1,)), ((), ())),
            preferred_element_type=jnp.float32))

    minvals = [_vq_minval(mm2s[i], x2s[i], c2row) for i in range(ub)]
    for i in range(ub):
        _vq_finish(mm2s[i], cb16, x2s[i], c2row, minvals[i],
                   out_ref, idx_ref, i)
    psum = jnp.zeros((), jnp.float32)
    for i in range(ub):
        psum += jnp.sum(minvals[i])

    @pl.when(g == 0)
    def _():
        loss_ref[...] = jnp.zeros((1, 1), jnp.float32)
    loss_ref[...] += psum.reshape(1, 1)


@jax.jit
def kernel(z, codebook):
    B, d, N = z.shape
    K = codebook.shape[0]

    out, idx3, loss_sum = pl.pallas_call(
        _vq_body,
        grid=(B // _UB,),
        in_specs=[
            pl.BlockSpec((_UB, d, N), lambda b: (b, 0, 0)),
            pl.BlockSpec((K, d), lambda b: (0, 0)),
            pl.BlockSpec((K, d), lambda b: (0, 0)),
        ],
        out_specs=[
            pl.BlockSpec((_UB, d, N), lambda b: (b, 0, 0)),
            pl.BlockSpec((_UB, 1, N), lambda b: (b, 0, 0)),
            pl.BlockSpec((1, 1), lambda b: (0, 0)),
        ],
        out_shape=[
            jax.ShapeDtypeStruct((B, d, N), jnp.float32),
            jax.ShapeDtypeStruct((B, 1, N), jnp.int32),
            jax.ShapeDtypeStruct((1, 1), jnp.float32),
        ],
    )(z, codebook, (codebook * -2.0).astype(jnp.bfloat16))

    commit_loss = 0.25 * loss_sum[0, 0] / (B * N * d)
    return out, idx3.reshape(B, N), commit_loss
